# baseline (speedup 1.0000x reference)
import math
from functools import partial

import numpy as np
import jax
import jax.numpy as jnp
from jax.experimental import pallas as pl
from jax.experimental.pallas import tpu as pltpu


def _bilinear_matrix(in_size: int, out_size: int) -> np.ndarray:
    scale = in_size / out_size
    src = (np.arange(out_size, dtype=np.float64) + 0.5) * scale - 0.5
    src = np.maximum(src, 0.0)
    i0 = np.minimum(np.floor(src).astype(np.int64), in_size - 1)
    i1 = np.minimum(i0 + 1, in_size - 1)
    frac = src - i0
    m = np.zeros((out_size, in_size), dtype=np.float64)
    rows = np.arange(out_size)
    m[rows, i0] += 1.0 - frac
    m[rows, i1] += frac
    return m


def _staged_bilinear(size: int, multiplier: float, n_stages: int) -> np.ndarray:
    m = np.eye(size, dtype=np.float64)
    cur = size
    for _ in range(n_stages):
        nxt = int(math.floor(cur * multiplier))
        m = _bilinear_matrix(cur, nxt) @ m
        cur = nxt
    return m


def _compute_chunk(xr, awt, ah, w_ref, b_ref, or_, *, SB, C, H, Ho, Cout):
    W = xr.shape[2]
    x = xr[...].reshape(SB * C * H, W).astype(jnp.bfloat16)
    y = jnp.dot(x, awt, preferred_element_type=jnp.float32)
    yh = y.astype(jnp.bfloat16)
    for b in range(SB):
        z = [jnp.dot(ah, yh[(b * C + c) * H:(b * C + c + 1) * H, :],
                     preferred_element_type=jnp.float32)
             for c in range(C)]
        for co in range(Cout):
            acc = z[0] * w_ref[co, 0]
            for c in range(1, C):
                acc = acc + z[c] * w_ref[co, c]
            or_[b, co * Ho:(co + 1) * Ho, :] = acc + b_ref[co]


def _pipeline_body(x_hbm, awt_ref, ah_ref, w_ref, b_ref, o_hbm,
                   x_buf, o_buf, in_sem, out_sem,
                   *, NS, SB, NBUF, C, H, Ho, Cout):
    p = pl.program_id(0)
    base0 = p * (NS * SB)

    def dma_in(slot, step):
        pltpu.make_async_copy(x_hbm.at[pl.ds(base0 + step * SB, SB)],
                              x_buf.at[slot], in_sem.at[slot]).start()

    def wait_in(slot):
        pltpu.make_async_copy(x_hbm.at[pl.ds(0, SB)],
                              x_buf.at[slot], in_sem.at[slot]).wait()

    def dma_out(slot, step):
        pltpu.make_async_copy(o_buf.at[slot],
                              o_hbm.at[pl.ds(base0 + step * SB, SB)],
                              out_sem.at[slot]).start()

    def wait_out(slot):
        pltpu.make_async_copy(o_buf.at[slot],
                              o_hbm.at[pl.ds(0, SB)], out_sem.at[slot]).wait()

    awt = awt_ref[...]
    ah = ah_ref[...]

    for j in range(min(NBUF - 1, NS)):
        dma_in(j, j)

    compute = partial(_compute_chunk, SB=SB, C=C, H=H, Ho=Ho, Cout=Cout)
    for i in range(NS):
        slot = i % NBUF
        if i + NBUF - 1 < NS:
            dma_in((i + NBUF - 1) % NBUF, i + NBUF - 1)
        wait_in(slot)
        if i >= NBUF:
            wait_out(slot)
        compute(x_buf.at[slot], awt, ah, w_ref, b_ref, o_buf.at[slot])
        dma_out(slot, i)

    for j in range(min(NBUF, NS)):
        wait_out((NS - 1 - j) % NBUF)


def kernel(x, w_map, b_map):
    N, C, H, W = x.shape
    Cout = int(w_map.shape[0])
    a_h = _staged_bilinear(H, 0.5, 1)
    a_w = _staged_bilinear(W, 0.5, 1)
    Ho, Wo = a_h.shape[0], a_w.shape[0]

    awt = jnp.asarray(a_w.T.astype(np.float32)).astype(jnp.bfloat16)
    ah = jnp.asarray(a_h.astype(np.float32)).astype(jnp.bfloat16)

    NC = 2 if N % 2 == 0 else 1
    NB = N // NC
    SB = 2 if NB % 2 == 0 else 1
    NS = NB // SB
    NBUF = min(4, NS)

    x_in = x.reshape(N, C * H, W)
    out = pl.pallas_call(
        partial(_pipeline_body, NS=NS, SB=SB, NBUF=NBUF,
                C=C, H=H, Ho=Ho, Cout=Cout),
        out_shape=jax.ShapeDtypeStruct((N, Cout * Ho, Wo), x.dtype),
        grid=(NC,),
        in_specs=[
            pl.BlockSpec(memory_space=pl.ANY),
            pl.BlockSpec((W, Wo), lambda n: (0, 0)),
            pl.BlockSpec((Ho, H), lambda n: (0, 0)),
            pl.BlockSpec(memory_space=pltpu.SMEM),
            pl.BlockSpec(memory_space=pltpu.SMEM),
        ],
        out_specs=pl.BlockSpec(memory_space=pl.ANY),
        scratch_shapes=[
            pltpu.VMEM((NBUF, SB, C * H, W), x.dtype),
            pltpu.VMEM((NBUF, SB, Cout * Ho, Wo), x.dtype),
            pltpu.SemaphoreType.DMA((NBUF,)),
            pltpu.SemaphoreType.DMA((NBUF,)),
        ],
        compiler_params=pltpu.CompilerParams(
            dimension_semantics=("parallel",),
            vmem_limit_bytes=100 * 1024 * 1024,
        ),
    )(x_in, awt, ah, jnp.asarray(w_map, jnp.float32), jnp.asarray(b_map, jnp.float32))
    return out.reshape(N, Cout, Ho, Wo)
